# baseline (speedup 1.0000x reference)
import functools

import jax
import jax.numpy as jnp
from jax import lax
from jax.experimental import pallas as pl
from jax.experimental.pallas import tpu as pltpu
from jax.experimental.pallas import tpu_sc as plsc

KE = 64
L = 16
SUB = 512
GB = 3
BLK = 128
MARGIN = 1.0
N_ROWS = 1000000
LAST_BLOCK = (N_ROWS - 1) // BLK
TAIL_OFF = N_ROWS - BLK
CHUNK = 128


def _cdiv(a, b):
    return (a + b - 1) // b


@functools.lru_cache(maxsize=None)
def _make_k1(n_pos: int):
    info = plsc.get_sparse_core_info()
    nc, ns = info.num_cores, info.num_subcores
    nw = nc * ns
    n_ent = 4 * n_pos
    n_rel = 2 * n_pos
    e_per_w = n_ent // nw
    r_per_w = n_rel // nw
    n_out = 2 * n_ent + 2 * n_rel
    assert e_per_w % SUB == 0 and r_per_w % SUB == 0

    mesh = plsc.VectorSubcoreMesh(core_axis_name="c", subcore_axis_name="s")

    @functools.partial(
        pl.kernel,
        mesh=mesh,
        compiler_params=pltpu.CompilerParams(
            needs_layout_passes=False, use_tc_tiling_on_sc=True),
        out_type=jax.ShapeDtypeStruct((n_out, BLK), jnp.float32),
        scratch_types=[
            pltpu.VMEM((SUB + L,), jnp.int32),
            pltpu.VMEM((KE, GB * BLK), jnp.float32),
            pltpu.VMEM((KE, GB * BLK), jnp.float32),
            pltpu.VMEM((KE, GB * BLK), jnp.float32),
            pltpu.VMEM((KE, GB * BLK), jnp.float32),
            pltpu.VMEM((32, BLK), jnp.float32),
            pltpu.VMEM((32, BLK), jnp.float32),
            pltpu.SemaphoreType.DMA,
            pltpu.SemaphoreType.DMA,
            pltpu.SemaphoreType.DMA,
        ],
    )
    def k1(ek, rk, embT, covT, rembT, rcovT, tlE, tlC, tlR, tlV, out,
           ksv, blkA0, blkB0, blkA1, blkB1, outA, outB,
           sem_blk0, sem_blk1, sem_out):
        wid = lax.axis_index("s") * nc + lax.axis_index("c")
        lane = lax.broadcasted_iota(jnp.int32, (L,), 0)

        def fetch_group(tblA, tblB, tailA, tailB, bg, bA, bB, sem):
            all_normal = bg + GB <= LAST_BLOCK

            @pl.when(all_normal)
            def _():
                off = pl.multiple_of(bg * BLK, BLK)
                pltpu.async_copy(tblA.at[:, pl.ds(off, GB * BLK)], bA, sem)
                pltpu.async_copy(tblB.at[:, pl.ds(off, GB * BLK)], bB, sem)

            @pl.when(jnp.logical_not(all_normal))
            def _():
                for j in range(GB):
                    bc = jnp.minimum(bg + j, LAST_BLOCK)
                    for tbl, tail_tbl, dst in ((tblA, tailA, bA),
                                               (tblB, tailB, bB)):
                        @pl.when(bc < LAST_BLOCK)
                        def _():
                            off = pl.multiple_of(bc * BLK, BLK)
                            pltpu.async_copy(
                                tbl.at[:, pl.ds(off, BLK)],
                                dst.at[:, pl.ds(j * BLK, BLK)], sem)

                        @pl.when(bc >= LAST_BLOCK)
                        def _():
                            pltpu.async_copy(
                                tail_tbl,
                                dst.at[:, pl.ds(j * BLK, BLK)], sem)

        def drain_blocks(sem):
            for _ in range(2):
                pltpu.make_async_copy(
                    embT.at[:, pl.ds(0, GB * BLK)], blkA0, sem).wait()

        def table_pass(tblA, tblB, tailA, tailB, keys, hit_base, n_sub,
                       outA_base, outB_base):
            for sp in range(n_sub):
                sub_base = hit_base + sp * SUB
                pltpu.sync_copy(keys.at[pl.ds(sub_base, SUB)],
                                ksv.at[pl.ds(0, SUB)])
                ksv[pl.ds(SUB, L)] = jnp.full((L,), 0x7FFFFFF0, jnp.int32)
                bg0 = ksv[pl.ds(0, L)][0] >> 7
                ngroups = ((ksv[pl.ds(SUB - 1, L)][0] >> 7) - bg0) // GB + 1

                def extract(bg, bA, bB, ptr):
                    def cond(c):
                        p, r = c
                        return jnp.logical_and(
                            p < SUB, (r >> 7) < bg + GB)

                    def hit(c):
                        p, r = c
                        loff = jnp.minimum((r >> 7) << 7, TAIL_OFF)
                        l = jnp.full((L,),
                                     ((r >> 7) - bg) * BLK + (r - loff),
                                     jnp.int32)
                        slot = p & 31
                        for q in range(KE // L):
                            cq = lane + q * L
                            outA[slot, pl.ds(q * L, L)] = plsc.load_gather(
                                bA, [cq, l])
                            outB[slot, pl.ds(q * L, L)] = plsc.load_gather(
                                bB, [cq, l])
                        p = p + 1

                        @pl.when((p & 15) == 0)
                        def _():
                            @pl.when(p > 16)
                            def _():
                                for _ in range(2):
                                    pltpu.make_async_copy(
                                        out.at[pl.ds(0, L), :],
                                        outA.at[pl.ds(0, L), :],
                                        sem_out).wait()

                            half = pl.multiple_of(
                                (((p >> 4) & 1) ^ 1) * L, L)
                            dstp = pl.multiple_of(sub_base + p - L, L)
                            pltpu.async_copy(
                                outA.at[pl.ds(half, L), :],
                                out.at[pl.ds(outA_base + dstp, L), :],
                                sem_out)
                            pltpu.async_copy(
                                outB.at[pl.ds(half, L), :],
                                out.at[pl.ds(outB_base + dstp, L), :],
                                sem_out)

                        return (p, ksv[pl.ds(p, L)][0])

                    p, _ = lax.while_loop(
                        cond, hit, (ptr, ksv[pl.ds(ptr, L)][0]))
                    return p

                def fg(g, bA, bB, sem):
                    fetch_group(tblA, tblB, tailA, tailB, bg0 + g * GB,
                                bA, bB, sem)

                fg(0, blkA0, blkB0, sem_blk0)

                def pbody(gp, ptr):
                    g0 = gp * 2
                    drain_blocks(sem_blk0)

                    @pl.when(g0 + 1 < ngroups)
                    def _():
                        fg(g0 + 1, blkA1, blkB1, sem_blk1)

                    ptr = extract(bg0 + g0 * GB, blkA0, blkB0, ptr)

                    @pl.when(g0 + 1 < ngroups)
                    def _():
                        drain_blocks(sem_blk1)

                    @pl.when(g0 + 2 < ngroups)
                    def _():
                        fg(g0 + 2, blkA0, blkB0, sem_blk0)

                    ptr = extract(bg0 + (g0 + 1) * GB, blkA1, blkB1, ptr)
                    return ptr

                lax.fori_loop(0, (ngroups + 1) // 2, pbody, jnp.int32(0))
                for _ in range(2):
                    pltpu.make_async_copy(
                        out.at[pl.ds(0, L), :],
                        outA.at[pl.ds(0, L), :], sem_out).wait()

        table_pass(embT, covT, tlE, tlC, ek, wid * e_per_w, e_per_w // SUB,
                   0, n_ent + n_rel)
        table_pass(rembT, rcovT, tlR, tlV, rk, wid * r_per_w,
                   r_per_w // SUB, n_ent, 2 * n_ent + n_rel)

    return k1


@functools.lru_cache(maxsize=None)
def _make_k2(n_pos: int):
    info = plsc.get_sparse_core_info()
    nc, ns = info.num_cores, info.num_subcores
    nw = nc * ns
    per_w = n_pos // nw
    n_chunks = per_w // CHUNK
    n_rows = 8 * n_pos + 4 * n_pos
    assert per_w * nw == n_pos and n_chunks * CHUNK == per_w

    mesh = plsc.VectorSubcoreMesh(core_axis_name="c", subcore_axis_name="s")

    C2 = 64
    row_t = pltpu.VMEM((C2, BLK), jnp.float32)
    idx_t = pltpu.VMEM((C2,), jnp.int32)
    n_chunks2 = per_w // C2

    @functools.partial(
        pl.kernel,
        mesh=mesh,
        compiler_params=pltpu.CompilerParams(
            needs_layout_passes=False, use_tc_tiling_on_sc=True),
        out_type=jax.ShapeDtypeStruct((nw, L), jnp.float32),
        scratch_types=(
            [idx_t] * 12 + [row_t] * 12 + [
                pltpu.VMEM((C2,), jnp.float32),
                pltpu.VMEM((C2,), jnp.float32),
                pltpu.VMEM((L,), jnp.float32),
                pltpu.SemaphoreType.DMA,
                pltpu.SemaphoreType.DMA,
            ]),
    )
    def k2(ihm, irm, itm, interm, out,
           xh0, xr0, xt0, yh0, yr0, yt0, xh1, xr1, xt1, yh1, yr1, yt1,
           hm0, hv0, tm0, tv0, rm0, rv0, hm1, hv1, tm1, tv1, rm1, rv1,
           score_p, score_n, accv, semP, semN):
        wid = lax.axis_index("s") * nc + lax.axis_index("c")
        lane = lax.broadcasted_iota(jnp.int32, (L,), 0)
        half = n_rows // 2
        setP = ((xh0, xr0, xt0, yh0, yr0, yt0),
                (hm0, hv0, tm0, tv0, rm0, rv0), semP)
        setN = ((xh1, xr1, xt1, yh1, yr1, yt1),
                (hm1, hv1, tm1, tv1, rm1, rv1), semN)

        def fire(base, st):
            (xh, xr, xt, yh, yr, yt), (hm, hv, tm, tv, rm, rv), sem = st
            pltpu.sync_copy(ihm.at[pl.ds(base, C2)], xh)
            pltpu.sync_copy(irm.at[pl.ds(base, C2)], xr)
            pltpu.sync_copy(itm.at[pl.ds(base, C2)], xt)
            for g in range(C2 // L):
                s = pl.ds(g * L, L)
                yh[s] = xh[s] + half
                yt[s] = xt[s] + half
                yr[s] = xr[s] + half
            pltpu.async_copy(interm.at[xh], hm, sem)
            pltpu.async_copy(interm.at[xt], tm, sem)
            pltpu.async_copy(interm.at[xr], rm, sem)
            pltpu.async_copy(interm.at[yh], hv, sem)
            pltpu.async_copy(interm.at[yt], tv, sem)
            pltpu.async_copy(interm.at[yr], rv, sem)

        def drain(st):
            _, bufs, sem = st
            for b in bufs:
                pltpu.make_async_copy(
                    interm.at[pl.ds(0, C2), :], b, sem).wait()

        def triple_scores(st, score_ref):
            _, (hm, hv, tm, tv, rm, rv), _ = st
            for g in range(C2 // L):
                def tbody(ti, svec):
                    t = g * L + ti
                    acc = jnp.zeros((L,), jnp.float32)
                    for q in range(KE // L):
                        s = pl.ds(q * L, L)
                        ev = tv[t, s] + hv[t, s]
                        d = rm[t, s] - (tm[t, s] - hm[t, s])
                        d2 = d * d
                        vrv = rv[t, s]
                        num = ev * ev + vrv * vrv + d2 * (ev + vrv)
                        acc = acc + num / (vrv * ev)
                    return jnp.where(lane == ti, jnp.sum(acc), svec)

                score_ref[pl.ds(g * L, L)] = lax.fori_loop(
                    0, L, tbody, jnp.zeros((L,), jnp.float32), unroll=4)

        fire(wid * per_w, setP)
        fire(n_pos + wid * per_w, setN)

        def chunk_body(c, acc16):
            drain(setP)
            triple_scores(setP, score_p)

            @pl.when(c + 1 < n_chunks2)
            def _():
                fire(wid * per_w + (c + 1) * C2, setP)

            drain(setN)
            triple_scores(setN, score_n)

            @pl.when(c + 1 < n_chunks2)
            def _():
                fire(n_pos + wid * per_w + (c + 1) * C2, setN)

            for g in range(C2 // L):
                sp = score_p[pl.ds(g * L, L)]
                sn = score_n[pl.ds(g * L, L)]
                acc16 = acc16 + jnp.maximum((sp - sn) * 0.25 + MARGIN, 0.0)
            return acc16

        acc16 = lax.fori_loop(0, n_chunks2, chunk_body,
                              jnp.zeros((L,), jnp.float32))
        accv[...] = acc16
        pltpu.sync_copy(accv, out.at[wid])

    return k2


def kernel(posX, negX, entityEmbed, entityCovar, relationEmbed,
           relationCovar):
    n = posX.shape[0]
    n2 = 2 * n
    heads = jnp.concatenate([posX[:, 0], negX[:, 0]])
    rels = jnp.concatenate([posX[:, 1], negX[:, 1]])
    tails = jnp.concatenate([posX[:, 2], negX[:, 2]])

    ekeys = jnp.concatenate([heads, tails])
    ek_s, ep_s = lax.sort_key_val(ekeys, jnp.arange(2 * n2, dtype=jnp.int32))
    rk_s, rp_s = lax.sort_key_val(rels, jnp.arange(n2, dtype=jnp.int32))
    inv_e = jnp.zeros((2 * n2,), jnp.int32).at[ep_s].add(
        jnp.arange(2 * n2, dtype=jnp.int32))
    inv_r = jnp.zeros((n2,), jnp.int32).at[rp_s].add(
        jnp.arange(n2, dtype=jnp.int32))

    embT = jnp.transpose(entityEmbed)
    covT = jnp.transpose(entityCovar)
    rembT = jnp.transpose(relationEmbed)
    rcovT = jnp.transpose(relationCovar)

    tail = lambda t: lax.slice(t, (0, TAIL_OFF), (KE, N_ROWS))
    interm = _make_k1(n)(ek_s, rk_s, embT, covT, rembT, rcovT,
                         tail(embT), tail(covT), tail(rembT), tail(rcovT))

    ihm = inv_e[:n2]
    itm = inv_e[n2:]
    irm = inv_r + 2 * n2
    out = _make_k2(n)(ihm, irm, itm, interm)
    return jnp.sum(out) / n


# speedup vs baseline: 1.0212x; 1.0212x over previous
import functools

import jax
import jax.numpy as jnp
from jax import lax
from jax.experimental import pallas as pl
from jax.experimental.pallas import tpu as pltpu
from jax.experimental.pallas import tpu_sc as plsc

KE = 64
L = 16
SUB = 1024
GB = 3
BLK = 128
MARGIN = 1.0
N_ROWS = 1000000
LAST_BLOCK = (N_ROWS - 1) // BLK
TAIL_OFF = N_ROWS - BLK
CHUNK = 128


def _cdiv(a, b):
    return (a + b - 1) // b


@functools.lru_cache(maxsize=None)
def _make_k1(n_pos: int):
    info = plsc.get_sparse_core_info()
    nc, ns = info.num_cores, info.num_subcores
    nw = nc * ns
    n_ent = 4 * n_pos
    n_rel = 2 * n_pos
    e_per_w = n_ent // nw
    r_per_w = n_rel // nw
    n_out = 2 * n_ent + 2 * n_rel
    assert e_per_w % SUB == 0 and r_per_w % SUB == 0

    mesh = plsc.VectorSubcoreMesh(core_axis_name="c", subcore_axis_name="s")

    @functools.partial(
        pl.kernel,
        mesh=mesh,
        compiler_params=pltpu.CompilerParams(
            needs_layout_passes=False, use_tc_tiling_on_sc=True),
        out_type=jax.ShapeDtypeStruct((n_out, BLK), jnp.float32),
        scratch_types=[
            pltpu.VMEM((SUB + L,), jnp.int32),
            pltpu.VMEM((KE, GB * BLK), jnp.float32),
            pltpu.VMEM((KE, GB * BLK), jnp.float32),
            pltpu.VMEM((KE, GB * BLK), jnp.float32),
            pltpu.VMEM((KE, GB * BLK), jnp.float32),
            pltpu.VMEM((32, BLK), jnp.float32),
            pltpu.VMEM((32, BLK), jnp.float32),
            pltpu.SemaphoreType.DMA,
            pltpu.SemaphoreType.DMA,
            pltpu.SemaphoreType.DMA,
        ],
    )
    def k1(ek, rk, embT, covT, rembT, rcovT, tlE, tlC, tlR, tlV, out,
           ksv, blkA0, blkB0, blkA1, blkB1, outA, outB,
           sem_blk0, sem_blk1, sem_out):
        wid = lax.axis_index("s") * nc + lax.axis_index("c")
        lane = lax.broadcasted_iota(jnp.int32, (L,), 0)

        def fetch_group(tblA, tblB, tailA, tailB, bg, bA, bB, sem):
            all_normal = bg + GB <= LAST_BLOCK

            @pl.when(all_normal)
            def _():
                off = pl.multiple_of(bg * BLK, BLK)
                pltpu.async_copy(tblA.at[:, pl.ds(off, GB * BLK)], bA, sem)
                pltpu.async_copy(tblB.at[:, pl.ds(off, GB * BLK)], bB, sem)

            @pl.when(jnp.logical_not(all_normal))
            def _():
                for j in range(GB):
                    bc = jnp.minimum(bg + j, LAST_BLOCK)
                    for tbl, tail_tbl, dst in ((tblA, tailA, bA),
                                               (tblB, tailB, bB)):
                        @pl.when(bc < LAST_BLOCK)
                        def _():
                            off = pl.multiple_of(bc * BLK, BLK)
                            pltpu.async_copy(
                                tbl.at[:, pl.ds(off, BLK)],
                                dst.at[:, pl.ds(j * BLK, BLK)], sem)

                        @pl.when(bc >= LAST_BLOCK)
                        def _():
                            pltpu.async_copy(
                                tail_tbl,
                                dst.at[:, pl.ds(j * BLK, BLK)], sem)

        def drain_blocks(sem):
            for _ in range(2):
                pltpu.make_async_copy(
                    embT.at[:, pl.ds(0, GB * BLK)], blkA0, sem).wait()

        def table_pass(tblA, tblB, tailA, tailB, keys, hit_base, n_sub,
                       outA_base, outB_base):
            for sp in range(n_sub):
                sub_base = hit_base + sp * SUB
                pltpu.sync_copy(keys.at[pl.ds(sub_base, SUB)],
                                ksv.at[pl.ds(0, SUB)])
                ksv[pl.ds(SUB, L)] = jnp.full((L,), 0x7FFFFFF0, jnp.int32)
                bg0 = ksv[pl.ds(0, L)][0] >> 7
                ngroups = ((ksv[pl.ds(SUB - 1, L)][0] >> 7) - bg0) // GB + 1

                def extract(bg, bA, bB, ptr):
                    def cond(c):
                        p, r = c
                        return jnp.logical_and(
                            p < SUB, (r >> 7) < bg + GB)

                    def hit(c):
                        p, r = c
                        loff = jnp.minimum((r >> 7) << 7, TAIL_OFF)
                        l = jnp.full((L,),
                                     ((r >> 7) - bg) * BLK + (r - loff),
                                     jnp.int32)
                        slot = p & 31
                        for q in range(KE // L):
                            cq = lane + q * L
                            outA[slot, pl.ds(q * L, L)] = plsc.load_gather(
                                bA, [cq, l])
                            outB[slot, pl.ds(q * L, L)] = plsc.load_gather(
                                bB, [cq, l])
                        p = p + 1

                        @pl.when((p & 15) == 0)
                        def _():
                            @pl.when(p > 16)
                            def _():
                                for _ in range(2):
                                    pltpu.make_async_copy(
                                        out.at[pl.ds(0, L), :],
                                        outA.at[pl.ds(0, L), :],
                                        sem_out).wait()

                            half = pl.multiple_of(
                                (((p >> 4) & 1) ^ 1) * L, L)
                            dstp = pl.multiple_of(sub_base + p - L, L)
                            pltpu.async_copy(
                                outA.at[pl.ds(half, L), :],
                                out.at[pl.ds(outA_base + dstp, L), :],
                                sem_out)
                            pltpu.async_copy(
                                outB.at[pl.ds(half, L), :],
                                out.at[pl.ds(outB_base + dstp, L), :],
                                sem_out)

                        return (p, ksv[pl.ds(p, L)][0])

                    p, _ = lax.while_loop(
                        cond, hit, (ptr, ksv[pl.ds(ptr, L)][0]))
                    return p

                def fg(g, bA, bB, sem):
                    fetch_group(tblA, tblB, tailA, tailB, bg0 + g * GB,
                                bA, bB, sem)

                fg(0, blkA0, blkB0, sem_blk0)

                def pbody(gp, ptr):
                    g0 = gp * 2
                    drain_blocks(sem_blk0)

                    @pl.when(g0 + 1 < ngroups)
                    def _():
                        fg(g0 + 1, blkA1, blkB1, sem_blk1)

                    ptr = extract(bg0 + g0 * GB, blkA0, blkB0, ptr)

                    @pl.when(g0 + 1 < ngroups)
                    def _():
                        drain_blocks(sem_blk1)

                    @pl.when(g0 + 2 < ngroups)
                    def _():
                        fg(g0 + 2, blkA0, blkB0, sem_blk0)

                    ptr = extract(bg0 + (g0 + 1) * GB, blkA1, blkB1, ptr)
                    return ptr

                lax.fori_loop(0, (ngroups + 1) // 2, pbody, jnp.int32(0))
                for _ in range(2):
                    pltpu.make_async_copy(
                        out.at[pl.ds(0, L), :],
                        outA.at[pl.ds(0, L), :], sem_out).wait()

        table_pass(embT, covT, tlE, tlC, ek, wid * e_per_w, e_per_w // SUB,
                   0, n_ent + n_rel)
        table_pass(rembT, rcovT, tlR, tlV, rk, wid * r_per_w,
                   r_per_w // SUB, n_ent, 2 * n_ent + n_rel)

    return k1


@functools.lru_cache(maxsize=None)
def _make_k2(n_pos: int):
    info = plsc.get_sparse_core_info()
    nc, ns = info.num_cores, info.num_subcores
    nw = nc * ns
    per_w = n_pos // nw
    n_chunks = per_w // CHUNK
    n_rows = 8 * n_pos + 4 * n_pos
    assert per_w * nw == n_pos and n_chunks * CHUNK == per_w

    mesh = plsc.VectorSubcoreMesh(core_axis_name="c", subcore_axis_name="s")

    C2 = 64
    row_t = pltpu.VMEM((C2, BLK), jnp.float32)
    idx_t = pltpu.VMEM((C2,), jnp.int32)
    n_chunks2 = per_w // C2

    @functools.partial(
        pl.kernel,
        mesh=mesh,
        compiler_params=pltpu.CompilerParams(
            needs_layout_passes=False, use_tc_tiling_on_sc=True),
        out_type=jax.ShapeDtypeStruct((nw, L), jnp.float32),
        scratch_types=(
            [idx_t] * 12 + [row_t] * 12 + [
                pltpu.VMEM((C2,), jnp.float32),
                pltpu.VMEM((C2,), jnp.float32),
                pltpu.VMEM((L,), jnp.float32),
                pltpu.SemaphoreType.DMA,
                pltpu.SemaphoreType.DMA,
            ]),
    )
    def k2(ihm, irm, itm, interm, out,
           xh0, xr0, xt0, yh0, yr0, yt0, xh1, xr1, xt1, yh1, yr1, yt1,
           hm0, hv0, tm0, tv0, rm0, rv0, hm1, hv1, tm1, tv1, rm1, rv1,
           score_p, score_n, accv, semP, semN):
        wid = lax.axis_index("s") * nc + lax.axis_index("c")
        lane = lax.broadcasted_iota(jnp.int32, (L,), 0)
        half = n_rows // 2
        setP = ((xh0, xr0, xt0, yh0, yr0, yt0),
                (hm0, hv0, tm0, tv0, rm0, rv0), semP)
        setN = ((xh1, xr1, xt1, yh1, yr1, yt1),
                (hm1, hv1, tm1, tv1, rm1, rv1), semN)

        def fire(base, st):
            (xh, xr, xt, yh, yr, yt), (hm, hv, tm, tv, rm, rv), sem = st
            pltpu.sync_copy(ihm.at[pl.ds(base, C2)], xh)
            pltpu.sync_copy(irm.at[pl.ds(base, C2)], xr)
            pltpu.sync_copy(itm.at[pl.ds(base, C2)], xt)
            for g in range(C2 // L):
                s = pl.ds(g * L, L)
                yh[s] = xh[s] + half
                yt[s] = xt[s] + half
                yr[s] = xr[s] + half
            pltpu.async_copy(interm.at[xh], hm, sem)
            pltpu.async_copy(interm.at[xt], tm, sem)
            pltpu.async_copy(interm.at[xr], rm, sem)
            pltpu.async_copy(interm.at[yh], hv, sem)
            pltpu.async_copy(interm.at[yt], tv, sem)
            pltpu.async_copy(interm.at[yr], rv, sem)

        def drain(st):
            _, bufs, sem = st
            for b in bufs:
                pltpu.make_async_copy(
                    interm.at[pl.ds(0, C2), :], b, sem).wait()

        def triple_scores(st, score_ref):
            _, (hm, hv, tm, tv, rm, rv), _ = st
            for g in range(C2 // L):
                def tbody(ti, svec):
                    t = g * L + ti
                    acc = jnp.zeros((L,), jnp.float32)
                    for q in range(KE // L):
                        s = pl.ds(q * L, L)
                        ev = tv[t, s] + hv[t, s]
                        d = rm[t, s] - (tm[t, s] - hm[t, s])
                        d2 = d * d
                        vrv = rv[t, s]
                        num = ev * ev + vrv * vrv + d2 * (ev + vrv)
                        acc = acc + num / (vrv * ev)
                    return jnp.where(lane == ti, jnp.sum(acc), svec)

                score_ref[pl.ds(g * L, L)] = lax.fori_loop(
                    0, L, tbody, jnp.zeros((L,), jnp.float32), unroll=4)

        fire(wid * per_w, setP)
        fire(n_pos + wid * per_w, setN)

        def chunk_body(c, acc16):
            drain(setP)
            triple_scores(setP, score_p)

            @pl.when(c + 1 < n_chunks2)
            def _():
                fire(wid * per_w + (c + 1) * C2, setP)

            drain(setN)
            triple_scores(setN, score_n)

            @pl.when(c + 1 < n_chunks2)
            def _():
                fire(n_pos + wid * per_w + (c + 1) * C2, setN)

            for g in range(C2 // L):
                sp = score_p[pl.ds(g * L, L)]
                sn = score_n[pl.ds(g * L, L)]
                acc16 = acc16 + jnp.maximum((sp - sn) * 0.25 + MARGIN, 0.0)
            return acc16

        acc16 = lax.fori_loop(0, n_chunks2, chunk_body,
                              jnp.zeros((L,), jnp.float32))
        accv[...] = acc16
        pltpu.sync_copy(accv, out.at[wid])

    return k2


def kernel(posX, negX, entityEmbed, entityCovar, relationEmbed,
           relationCovar):
    n = posX.shape[0]
    n2 = 2 * n
    heads = jnp.concatenate([posX[:, 0], negX[:, 0]])
    rels = jnp.concatenate([posX[:, 1], negX[:, 1]])
    tails = jnp.concatenate([posX[:, 2], negX[:, 2]])

    ekeys = jnp.concatenate([heads, tails])
    ek_s, ep_s = lax.sort_key_val(ekeys, jnp.arange(2 * n2, dtype=jnp.int32))
    rk_s, rp_s = lax.sort_key_val(rels, jnp.arange(n2, dtype=jnp.int32))
    inv_e = jnp.zeros((2 * n2,), jnp.int32).at[ep_s].add(
        jnp.arange(2 * n2, dtype=jnp.int32))
    inv_r = jnp.zeros((n2,), jnp.int32).at[rp_s].add(
        jnp.arange(n2, dtype=jnp.int32))

    embT = jnp.transpose(entityEmbed)
    covT = jnp.transpose(entityCovar)
    rembT = jnp.transpose(relationEmbed)
    rcovT = jnp.transpose(relationCovar)

    tail = lambda t: lax.slice(t, (0, TAIL_OFF), (KE, N_ROWS))
    interm = _make_k1(n)(ek_s, rk_s, embT, covT, rembT, rcovT,
                         tail(embT), tail(covT), tail(rembT), tail(rcovT))

    ihm = inv_e[:n2]
    itm = inv_e[n2:]
    irm = inv_r + 2 * n2
    out = _make_k2(n)(ihm, irm, itm, interm)
    return jnp.sum(out) / n
